# Initial kernel scaffold; baseline (speedup 1.0000x reference)
import functools

import jax
import jax.numpy as jnp
from jax import lax
from jax.experimental import pallas as pl
from jax.experimental.pallas import tpu as pltpu
from jax.experimental.pallas import tpu_sc as plsc

_N = 10000
_D = 128
_NT = 10368
_NW = 32
_EPW = 10368
_EPAD = _EPW * _NW
_CH = 128
_NCHUNK = _EPW // _CH
_AROWS = 12288
_DUMMY = _N


def _sc_mesh():
    return plsc.VectorSubcoreMesh(core_axis_name="c", subcore_axis_name="s")


def _sc_degree(src, dst):
    nz = _NT // 16

    @functools.partial(
        pl.kernel,
        out_type=jax.ShapeDtypeStruct((2, 2, _NT), jnp.float32),
        mesh=_sc_mesh(),
        scratch_types=[
            pltpu.VMEM((_CH,), jnp.int32),
            pltpu.VMEM((_CH,), jnp.int32),
            pltpu.VMEM((_CH,), jnp.float32),
            pltpu.VMEM((nz,), jnp.float32),
            pltpu.VMEM_SHARED((_NT,), jnp.float32),
            pltpu.VMEM_SHARED((_NT,), jnp.float32),
        ],
    )
    def k(src_hbm, dst_hbm, out_hbm, sidx, didx, ones, zvec, hsrc, hdst):
        c = lax.axis_index("c")
        s = lax.axis_index("s")
        wid = c * 16 + s

        def fill(i, carry):
            ones[pl.ds(i * 16, 16)] = jnp.ones((16,), jnp.float32)
            return carry

        lax.fori_loop(0, _CH // 16, fill, 0)

        def zfill(i, carry):
            off = jnp.minimum(i * 16, nz - 16)
            zvec[pl.ds(off, 16)] = jnp.zeros((16,), jnp.float32)
            return carry

        lax.fori_loop(0, (nz + 15) // 16, zfill, 0)
        pltpu.sync_copy(zvec, hsrc.at[pl.ds(s * nz, nz)])
        pltpu.sync_copy(zvec, hdst.at[pl.ds(s * nz, nz)])
        plsc.subcore_barrier()

        def body(j, carry):
            base = wid * _EPW + j * _CH
            pltpu.sync_copy(src_hbm.at[pl.ds(base, _CH)], sidx)
            pltpu.sync_copy(dst_hbm.at[pl.ds(base, _CH)], didx)
            pltpu.sync_copy(ones, hsrc.at[sidx], add=True)
            pltpu.sync_copy(ones, hdst.at[didx], add=True)
            return carry

        lax.fori_loop(0, _NCHUNK, body, 0)
        plsc.subcore_barrier()
        pltpu.sync_copy(hsrc.at[pl.ds(s * nz, nz)],
                        out_hbm.at[c, 0, pl.ds(s * nz, nz)])
        pltpu.sync_copy(hdst.at[pl.ds(s * nz, nz)],
                        out_hbm.at[c, 1, pl.ds(s * nz, nz)])

    return k(src, dst)


def _sc_aggregate(hs, src, dst):
    rpt = _AROWS // 16
    orow = _NT // 16

    @functools.partial(
        pl.kernel,
        out_type=jax.ShapeDtypeStruct((2, _NT, _D), jnp.float32),
        mesh=_sc_mesh(),
        scratch_types=[
            pltpu.VMEM((_CH,), jnp.int32),
            pltpu.VMEM((_CH,), jnp.int32),
            pltpu.VMEM((_CH, _D), jnp.float32),
            pltpu.VMEM((_CH, _D), jnp.float32),
            pltpu.VMEM_SHARED((_AROWS, _D), jnp.float32),
            pltpu.SemaphoreType.DMA,
        ],
    )
    def k(hs_hbm, src_hbm, dst_hbm, out_hbm, sidx, didx, rows, zbuf, acc, sem):
        c = lax.axis_index("c")
        s = lax.axis_index("s")
        wid = c * 16 + s

        def zfill(i, carry):
            for j in range(_D // 16):
                zbuf[i, pl.ds(j * 16, 16)] = jnp.zeros((16,), jnp.float32)
            return carry

        lax.fori_loop(0, _CH, zfill, 0)
        for t in range(rpt // _CH):
            pltpu.sync_copy(zbuf, acc.at[pl.ds(s * rpt + t * _CH, _CH), :])
        plsc.subcore_barrier()

        def body(j, carry):
            base = wid * _EPW + j * _CH
            pltpu.sync_copy(src_hbm.at[pl.ds(base, _CH)], sidx)
            pltpu.sync_copy(dst_hbm.at[pl.ds(base, _CH)], didx)
            pltpu.async_copy(hs_hbm.at[sidx], rows, sem).wait()
            pltpu.sync_copy(rows, acc.at[didx], add=True)
            return carry

        lax.fori_loop(0, _NCHUNK, body, 0)
        plsc.subcore_barrier()
        pltpu.sync_copy(acc.at[pl.ds(s * orow, orow), :],
                        out_hbm.at[c, pl.ds(s * orow, orow), :])

    return k(hs, src, dst)


_R = 1296


def _tc_prep(degT, feat_pad):
    def body(deg_ref, feat_ref, hs_ref, ns_ref, nd_ref):
        deg = deg_ref[...]
        dsrc = deg[:, 0:1] + deg[:, 2:3]
        ddst = deg[:, 1:2] + deg[:, 3:4]
        ns = lax.rsqrt(jnp.maximum(dsrc, 1.0))
        nd = lax.rsqrt(jnp.maximum(ddst, 1.0))
        ns_ref[...] = ns
        nd_ref[...] = nd
        hs_ref[...] = feat_ref[...] * ns

    return pl.pallas_call(
        body,
        grid=(_NT // _R,),
        in_specs=[pl.BlockSpec((_R, 4), lambda i: (i, 0)),
                  pl.BlockSpec((_R, _D), lambda i: (i, 0))],
        out_specs=[pl.BlockSpec((_R, _D), lambda i: (i, 0)),
                   pl.BlockSpec((_R, 1), lambda i: (i, 0)),
                   pl.BlockSpec((_R, 1), lambda i: (i, 0))],
        out_shape=[jax.ShapeDtypeStruct((_NT, _D), jnp.float32),
                   jax.ShapeDtypeStruct((_NT, 1), jnp.float32),
                   jax.ShapeDtypeStruct((_NT, 1), jnp.float32)],
    )(degT, feat_pad)


def _tc_layer(aggp, nd, ns, W, b, apply_relu):
    def body(aggp_ref, nd_ref, ns_ref, w_ref, b_ref, out_ref):
        t = aggp_ref[0] + aggp_ref[1]
        z = jnp.dot(t, w_ref[...], preferred_element_type=jnp.float32)
        y = z * nd_ref[...] + b_ref[...]
        if apply_relu:
            y = jnp.maximum(y, 0.0) * ns_ref[...]
        out_ref[...] = y

    return pl.pallas_call(
        body,
        grid=(_NT // _R,),
        in_specs=[pl.BlockSpec((2, _R, _D), lambda i: (0, i, 0)),
                  pl.BlockSpec((_R, 1), lambda i: (i, 0)),
                  pl.BlockSpec((_R, 1), lambda i: (i, 0)),
                  pl.BlockSpec((_D, _D), lambda i: (0, 0)),
                  pl.BlockSpec((1, _D), lambda i: (0, 0))],
        out_specs=pl.BlockSpec((_R, _D), lambda i: (i, 0)),
        out_shape=jax.ShapeDtypeStruct((_NT, _D), jnp.float32),
    )(aggp, nd, ns, W, b)


def kernel(feat, edge_index, W1, b1, W2, b2, W3, b3):
    ei = edge_index.astype(jnp.int32)
    loops = jnp.arange(_N, dtype=jnp.int32)
    src = jnp.concatenate([ei[0], loops])
    dst = jnp.concatenate([ei[1], loops])
    npad = _EPAD - src.shape[0]
    padv = jnp.full((npad,), _DUMMY, jnp.int32)
    src = jnp.concatenate([src, padv])
    dst = jnp.concatenate([dst, padv])
    feat_pad = jnp.pad(feat.astype(jnp.float32), ((0, _NT - _N), (0, 0)))

    degp = _sc_degree(src, dst)
    degT = jnp.transpose(degp, (2, 0, 1)).reshape(_NT, 4)
    hs, ns, nd = _tc_prep(degT, feat_pad)

    for W, b, last in ((W1, b1, False), (W2, b2, False), (W3, b3, True)):
        aggp = _sc_aggregate(hs, src, dst)
        hs = _tc_layer(aggp, nd, ns, W.astype(jnp.float32),
                       b.astype(jnp.float32).reshape(1, _D), not last)
    return hs[:_N]


# speedup vs baseline: 6.9526x; 6.9526x over previous
import functools

import jax
import jax.numpy as jnp
from jax import lax
from jax.experimental import pallas as pl
from jax.experimental.pallas import tpu as pltpu
from jax.experimental.pallas import tpu_sc as plsc

_N = 10000
_D = 128
_NT = 10240
_NW = 32
_EPW = 10368
_EPAD = _EPW * _NW
_CH = 128
_NCHUNK = _EPW // _CH
_AROWS = _NT
_DUMMY = _N


def _sc_mesh():
    return plsc.VectorSubcoreMesh(core_axis_name="c", subcore_axis_name="s")


def _sc_degree(src, dst):
    nz = _NT // 16

    @functools.partial(
        pl.kernel,
        out_type=[jax.ShapeDtypeStruct((2 * _NT,), jnp.float32),
                  jax.ShapeDtypeStruct((2 * _NT,), jnp.float32)],
        mesh=_sc_mesh(),
        scratch_types=[
            pltpu.VMEM((_CH,), jnp.int32),
            pltpu.VMEM((_CH,), jnp.int32),
            pltpu.VMEM((_CH,), jnp.float32),
            pltpu.VMEM((nz,), jnp.float32),
            pltpu.VMEM_SHARED((_NT,), jnp.float32),
            pltpu.VMEM_SHARED((_NT,), jnp.float32),
        ],
    )
    def k(src_hbm, dst_hbm, osrc_hbm, odst_hbm, sidx, didx, ones, zvec,
          hsrc, hdst):
        c = lax.axis_index("c")
        s = lax.axis_index("s")
        wid = c * 16 + s

        def fill(i, carry):
            ones[pl.ds(i * 16, 16)] = jnp.ones((16,), jnp.float32)
            return carry

        lax.fori_loop(0, _CH // 16, fill, 0)

        def zfill(i, carry):
            zvec[pl.ds(i * 16, 16)] = jnp.zeros((16,), jnp.float32)
            return carry

        lax.fori_loop(0, nz // 16, zfill, 0)
        pltpu.sync_copy(zvec, hsrc.at[pl.ds(s * nz, nz)])
        pltpu.sync_copy(zvec, hdst.at[pl.ds(s * nz, nz)])
        plsc.subcore_barrier()

        def body(j, carry):
            base = wid * _EPW + j * _CH
            pltpu.sync_copy(src_hbm.at[pl.ds(base, _CH)], sidx)
            pltpu.sync_copy(dst_hbm.at[pl.ds(base, _CH)], didx)
            pltpu.sync_copy(ones, hsrc.at[sidx], add=True)
            pltpu.sync_copy(ones, hdst.at[didx], add=True)
            return carry

        lax.fori_loop(0, _NCHUNK, body, 0)
        plsc.subcore_barrier()
        pltpu.sync_copy(hsrc.at[pl.ds(s * nz, nz)],
                        osrc_hbm.at[pl.ds(c * _NT + s * nz, nz)])
        pltpu.sync_copy(hdst.at[pl.ds(s * nz, nz)],
                        odst_hbm.at[pl.ds(c * _NT + s * nz, nz)])

    return k(src, dst)


def _sc_aggregate(hs, src, dst):
    rpt = _AROWS // 16
    orow = _NT // 16

    @functools.partial(
        pl.kernel,
        out_type=jax.ShapeDtypeStruct((2, _NT, _D), jnp.float32),
        mesh=_sc_mesh(),
        scratch_types=[
            pltpu.VMEM((_CH,), jnp.int32),
            pltpu.VMEM((_CH,), jnp.int32),
            pltpu.VMEM((_CH, _D), jnp.float32),
            pltpu.VMEM((_CH, _D), jnp.float32),
            pltpu.VMEM_SHARED((_AROWS, _D), jnp.float32),
            pltpu.SemaphoreType.DMA,
        ],
    )
    def k(hs_hbm, src_hbm, dst_hbm, out_hbm, sidx, didx, rows, zbuf, acc, sem):
        c = lax.axis_index("c")
        s = lax.axis_index("s")
        wid = c * 16 + s

        def zfill(i, carry):
            for j in range(_D // 16):
                zbuf[i, pl.ds(j * 16, 16)] = jnp.zeros((16,), jnp.float32)
            return carry

        lax.fori_loop(0, _CH, zfill, 0)
        for t in range(rpt // _CH):
            pltpu.sync_copy(zbuf, acc.at[pl.ds(s * rpt + t * _CH, _CH), :])
        plsc.subcore_barrier()

        def body(j, carry):
            base = wid * _EPW + j * _CH
            pltpu.sync_copy(src_hbm.at[pl.ds(base, _CH)], sidx)
            pltpu.sync_copy(dst_hbm.at[pl.ds(base, _CH)], didx)
            pltpu.async_copy(hs_hbm.at[sidx], rows, sem).wait()
            pltpu.sync_copy(rows, acc.at[didx], add=True)
            return carry

        lax.fori_loop(0, _NCHUNK, body, 0)
        plsc.subcore_barrier()
        pltpu.sync_copy(acc.at[pl.ds(s * orow, orow), :],
                        out_hbm.at[c, pl.ds(s * orow, orow), :])

    return k(hs, src, dst)


_R = 1280


def _tc_prep(degT, feat_pad):
    def body(deg_ref, feat_ref, hs_ref, ns_ref, nd_ref):
        deg = deg_ref[...]
        dsrc = deg[:, 0:1] + deg[:, 2:3]
        ddst = deg[:, 1:2] + deg[:, 3:4]
        ns = lax.rsqrt(jnp.maximum(dsrc, 1.0))
        nd = lax.rsqrt(jnp.maximum(ddst, 1.0))
        ns_ref[...] = ns
        nd_ref[...] = nd
        hs_ref[...] = feat_ref[...] * ns

    return pl.pallas_call(
        body,
        grid=(_NT // _R,),
        in_specs=[pl.BlockSpec((_R, 4), lambda i: (i, 0)),
                  pl.BlockSpec((_R, _D), lambda i: (i, 0))],
        out_specs=[pl.BlockSpec((_R, _D), lambda i: (i, 0)),
                   pl.BlockSpec((_R, 1), lambda i: (i, 0)),
                   pl.BlockSpec((_R, 1), lambda i: (i, 0))],
        out_shape=[jax.ShapeDtypeStruct((_NT, _D), jnp.float32),
                   jax.ShapeDtypeStruct((_NT, 1), jnp.float32),
                   jax.ShapeDtypeStruct((_NT, 1), jnp.float32)],
    )(degT, feat_pad)


def _tc_layer(aggp, nd, ns, W, b, apply_relu):
    def body(aggp_ref, nd_ref, ns_ref, w_ref, b_ref, out_ref):
        t = aggp_ref[0] + aggp_ref[1]
        z = jnp.dot(t, w_ref[...], preferred_element_type=jnp.float32)
        y = z * nd_ref[...] + b_ref[...]
        if apply_relu:
            y = jnp.maximum(y, 0.0) * ns_ref[...]
        out_ref[...] = y

    return pl.pallas_call(
        body,
        grid=(_NT // _R,),
        in_specs=[pl.BlockSpec((2, _R, _D), lambda i: (0, i, 0)),
                  pl.BlockSpec((_R, 1), lambda i: (i, 0)),
                  pl.BlockSpec((_R, 1), lambda i: (i, 0)),
                  pl.BlockSpec((_D, _D), lambda i: (0, 0)),
                  pl.BlockSpec((1, _D), lambda i: (0, 0))],
        out_specs=pl.BlockSpec((_R, _D), lambda i: (i, 0)),
        out_shape=jax.ShapeDtypeStruct((_NT, _D), jnp.float32),
    )(aggp, nd, ns, W, b)


def kernel(feat, edge_index, W1, b1, W2, b2, W3, b3):
    ei = edge_index.astype(jnp.int32)
    loops = jnp.arange(_N, dtype=jnp.int32)
    src = jnp.concatenate([ei[0], loops])
    dst = jnp.concatenate([ei[1], loops])
    npad = _EPAD - src.shape[0]
    padv = jnp.full((npad,), _DUMMY, jnp.int32)
    src = jnp.concatenate([src, padv])
    dst = jnp.concatenate([dst, padv])
    feat_pad = jnp.pad(feat.astype(jnp.float32), ((0, _NT - _N), (0, 0)))

    dsrc, ddst = _sc_degree(src, dst)
    degT = jnp.stack([dsrc[:_NT], ddst[:_NT], dsrc[_NT:], ddst[_NT:]],
                     axis=1)
    hs, ns, nd = _tc_prep(degT, feat_pad)

    for W, b, last in ((W1, b1, False), (W2, b2, False), (W3, b3, True)):
        aggp = _sc_aggregate(hs, src, dst)
        hs = _tc_layer(aggp, nd, ns, W.astype(jnp.float32),
                       b.astype(jnp.float32).reshape(1, _D), not last)
    return hs[:_N]
